# Initial kernel scaffold; baseline (speedup 1.0000x reference)
import jax
import jax.numpy as jnp
from jax.experimental import pallas as pl


def kernel(features, edge_index, batch, W, b):
    raise NotImplementedError("write your pallas kernel here")


# speedup vs baseline: 13.8706x; 13.8706x over previous
import functools

import jax
import jax.numpy as jnp
from jax import lax
from jax.experimental import pallas as pl
from jax.experimental.pallas import tpu as pltpu
from jax.experimental.pallas import tpu_sc as plsc

NC = 2
NS = 16
NW = NC * NS
L = 16
K = 128


def _deg_kernel(nrows, epw, cpw):
    rps = nrows // NS
    zr = 64
    mesh = plsc.VectorSubcoreMesh(core_axis_name="c", subcore_axis_name="s")

    @functools.partial(
        pl.kernel,
        mesh=mesh,
        out_type=jax.ShapeDtypeStruct((NC, nrows, L), jnp.float32),
        scratch_types=[
            pltpu.VMEM_SHARED((nrows, L), jnp.float32),
            pltpu.VMEM((K, L), jnp.float32),
            pltpu.VMEM((zr, L), jnp.float32),
            pltpu.VMEM((K,), jnp.int32),
        ],
    )
    def body(dst_hbm, out_hbm, deg_sh, ones_v, z_v, idx_v):
        c = lax.axis_index("c")
        s = lax.axis_index("s")
        wid = c * NS + s
        for i in range(K):
            ones_v[i, :] = jnp.ones((L,), jnp.float32)
        for i in range(zr):
            z_v[i, :] = jnp.zeros((L,), jnp.float32)

        @pl.loop(0, rps // zr)
        def _zero(r):
            pltpu.sync_copy(z_v, deg_sh.at[pl.ds(s * rps + r * zr, zr)])

        plsc.subcore_barrier()

        base0 = wid * epw

        @pl.loop(0, cpw)
        def _hist(j):
            pltpu.sync_copy(dst_hbm.at[pl.ds(base0 + j * K, K)], idx_v)
            pltpu.sync_copy(ones_v, deg_sh.at[idx_v], add=True)

        plsc.subcore_barrier()
        pltpu.sync_copy(
            deg_sh.at[pl.ds(s * rps, rps)], out_hbm.at[c, pl.ds(s * rps, rps)]
        )

    return body


def _mp_kernel(nrows, d, epw, cpw):
    rps = nrows // NS
    zr = 64
    mesh = plsc.VectorSubcoreMesh(core_axis_name="c", subcore_axis_name="s")

    @functools.partial(
        pl.kernel,
        mesh=mesh,
        out_type=jax.ShapeDtypeStruct((NC, nrows, d), jnp.float32),
        scratch_types=[
            pltpu.VMEM_SHARED((nrows, d), jnp.float32),
            pltpu.VMEM((K, d), jnp.float32),
            pltpu.VMEM((K,), jnp.int32),
            pltpu.VMEM((K,), jnp.int32),
            pltpu.SemaphoreType.DMA,
        ],
    )
    def body(y_hbm, src_hbm, dst_hbm, out_hbm, acc_sh, rows_v, sidx_v, didx_v, sem):
        c = lax.axis_index("c")
        s = lax.axis_index("s")
        wid = c * NS + s
        for i in range(zr):
            for k in range(d // L):
                rows_v[i, pl.ds(k * L, L)] = jnp.zeros((L,), jnp.float32)

        @pl.loop(0, rps // zr)
        def _zero(r):
            pltpu.sync_copy(rows_v.at[pl.ds(0, zr)], acc_sh.at[pl.ds(s * rps + r * zr, zr)])

        plsc.subcore_barrier()

        base0 = wid * epw

        @pl.loop(0, cpw)
        def _edges(j):
            base = base0 + j * K
            pltpu.sync_copy(src_hbm.at[pl.ds(base, K)], sidx_v)
            pltpu.sync_copy(dst_hbm.at[pl.ds(base, K)], didx_v)
            pltpu.async_copy(y_hbm.at[sidx_v], rows_v, sem).wait()
            pltpu.sync_copy(rows_v, acc_sh.at[didx_v], add=True)

        plsc.subcore_barrier()
        pltpu.sync_copy(
            acc_sh.at[pl.ds(s * rps, rps)], out_hbm.at[c, pl.ds(s * rps, rps)]
        )

    return body


def _linear_body(feat_ref, w_ref, degt_ref, y_ref, dis_ref):
    deg = degt_ref[:, 0:1] + degt_ref[:, 1:2] + 1.0
    dis = lax.rsqrt(deg)
    x = jnp.dot(feat_ref[...], w_ref[...], preferred_element_type=jnp.float32)
    y_ref[...] = x * dis
    dis_ref[...] = dis


def _combine_body(n, q_ref, y_ref, dis_ref, b_ref, out_ref):
    ssum = q_ref[0, :n, :] + q_ref[1, :n, :] + y_ref[...]
    out_ref[...] = ssum * dis_ref[...] + b_ref[...][None, :]


def kernel(features, edge_index, batch, W, b):
    n, d = features.shape
    e = edge_index.shape[1]

    epw = -(-e // (NW * K)) * K
    cpw = epw // K
    e_pad = epw * NW
    rps = -(-(n + 1) // (NS * 64)) * 64
    nrows = rps * NS

    src = edge_index[0].astype(jnp.int32)
    dst = edge_index[1].astype(jnp.int32)
    pad = e_pad - e
    src_p = jnp.concatenate([src, jnp.zeros((pad,), jnp.int32)])
    dst_p = jnp.concatenate([dst, jnp.full((pad,), n, jnp.int32)])

    degp = _deg_kernel(nrows, epw, cpw)(dst_p)
    degt = jnp.stack([degp[0, :n, 0], degp[1, :n, 0]], axis=1)

    y, dis = pl.pallas_call(
        _linear_body,
        out_shape=[
            jax.ShapeDtypeStruct((n, d), jnp.float32),
            jax.ShapeDtypeStruct((n, 1), jnp.float32),
        ],
    )(features, W, degt)

    q = _mp_kernel(nrows, d, epw, cpw)(y, src_p, dst_p)

    out = pl.pallas_call(
        functools.partial(_combine_body, n),
        out_shape=jax.ShapeDtypeStruct((n, d), jnp.float32),
    )(q, y, dis, b)
    return out
